# Initial kernel scaffold; baseline (speedup 1.0000x reference)
import jax
import jax.numpy as jnp
from jax.experimental import pallas as pl


def kernel(src_table, trg_table, src_indices, trg_indices):
    raise NotImplementedError("write your pallas kernel here")


# speedup vs baseline: 2.0564x; 2.0564x over previous
import functools

import jax
import jax.numpy as jnp
from jax import lax
from jax.experimental import pallas as pl
from jax.experimental.pallas import tpu as pltpu
from jax.experimental.pallas import tpu_sc as plsc

D = 256
NC, NS = 2, 16
NW = NC * NS
CHUNK = 128


def _emb_body(src_tab, trg_tab, src_idx, trg_idx, src_out, trg_out,
              idx_v, rows_v, gsem):
    B = src_idx.shape[0]
    bpw = B // NW
    nch = bpw // CHUNK
    wid = lax.axis_index("s") * NC + lax.axis_index("c")
    base = wid * bpw
    for idx_hbm, tab_hbm, out_hbm in ((src_idx, src_tab, src_out),
                                      (trg_idx, trg_tab, trg_out)):
        def body(c, _, idx_hbm=idx_hbm, tab_hbm=tab_hbm, out_hbm=out_hbm):
            off = base + c * CHUNK
            pltpu.sync_copy(idx_hbm.at[pl.ds(off, CHUNK)], idx_v)
            pltpu.async_copy(tab_hbm.at[idx_v], rows_v, gsem).wait()
            pltpu.sync_copy(rows_v, out_hbm.at[pl.ds(off, CHUNK)])
            return 0
        lax.fori_loop(0, nch, body, 0)


def kernel(src_table, trg_table, src_indices, trg_indices):
    Bt, S = src_indices.shape
    B = Bt * S
    si = src_indices.reshape(B)
    ti = trg_indices.reshape(B)
    mesh = plsc.VectorSubcoreMesh(core_axis_name="c", subcore_axis_name="s",
                                  num_cores=NC, num_subcores=NS)
    k = pl.kernel(
        _emb_body,
        out_type=(jax.ShapeDtypeStruct((B, D), jnp.float32),
                  jax.ShapeDtypeStruct((B, D), jnp.float32)),
        mesh=mesh,
        scratch_types=[
            pltpu.VMEM((CHUNK,), jnp.int32),
            pltpu.VMEM((CHUNK, D), jnp.float32),
            pltpu.SemaphoreType.DMA,
        ],
    )
    src_out, trg_out = k(src_table, trg_table, si, ti)
    return (src_out.reshape(Bt, S, D), trg_out.reshape(Bt, S, D))


# speedup vs baseline: 2.9035x; 1.4120x over previous
import jax
import jax.numpy as jnp
from jax import lax
from jax.experimental import pallas as pl
from jax.experimental.pallas import tpu as pltpu
from jax.experimental.pallas import tpu_sc as plsc

D = 256
NC, NS = 2, 16
NW = NC * NS
CHUNK = 80
NBUF = 5


def _emb_body(src_tab, trg_tab, src_idx, trg_idx, src_out, trg_out,
              idx_v, rows_v, *sems):
    gsem = sems[:NBUF]
    osem = sems[NBUF:]
    B = src_idx.shape[0]
    bpw = B // NW
    nch = bpw // CHUNK
    nr = nch // NBUF
    wid = lax.axis_index("s") * NC + lax.axis_index("c")
    base = wid * bpw

    for idx_hbm, tab_hbm, out_hbm in ((src_idx, src_tab, src_out),
                                      (trg_idx, trg_tab, trg_out)):

        def fetch(b, c):
            off = base + c * CHUNK
            pltpu.sync_copy(idx_hbm.at[pl.ds(off, CHUNK)], idx_v.at[b])
            pltpu.async_copy(tab_hbm.at[idx_v.at[b]], rows_v.at[b], gsem[b])

        def gwait(b):
            pltpu.make_async_copy(tab_hbm.at[idx_v.at[b]], rows_v.at[b],
                                  gsem[b]).wait()

        def store(b, c):
            off = base + c * CHUNK
            pltpu.async_copy(rows_v.at[b], out_hbm.at[pl.ds(off, CHUNK)],
                             osem[b])

        def owait(b, c):
            off = base + c * CHUNK
            pltpu.make_async_copy(rows_v.at[b], out_hbm.at[pl.ds(off, CHUNK)],
                                  osem[b]).wait()

        for b in range(NBUF):
            fetch(b, b)

        def round_body(r, _):
            g = r * NBUF
            for b in range(NBUF):
                gwait(b)
                store(b, g + b)
            for b in range(NBUF):
                owait(b, g + b)
                fetch(b, g + b + NBUF)
            return 0

        lax.fori_loop(0, nr - 1, round_body, 0)

        g = (nr - 1) * NBUF
        for b in range(NBUF):
            gwait(b)
            store(b, g + b)
        for b in range(NBUF):
            owait(b, g + b)


def kernel(src_table, trg_table, src_indices, trg_indices):
    Bt, S = src_indices.shape
    B = Bt * S
    si = src_indices.reshape(B)
    ti = trg_indices.reshape(B)
    mesh = plsc.VectorSubcoreMesh(core_axis_name="c", subcore_axis_name="s",
                                  num_cores=NC, num_subcores=NS)
    k = pl.kernel(
        _emb_body,
        out_type=(jax.ShapeDtypeStruct((B, D), jnp.float32),
                  jax.ShapeDtypeStruct((B, D), jnp.float32)),
        mesh=mesh,
        scratch_types=(
            [pltpu.VMEM((NBUF, CHUNK), jnp.int32),
             pltpu.VMEM((NBUF, CHUNK, D), jnp.float32)]
            + [pltpu.SemaphoreType.DMA] * (2 * NBUF)
        ),
    )
    src_out, trg_out = k(src_table, trg_table, si, ti)
    return (src_out.reshape(Bt, S, D), trg_out.reshape(Bt, S, D))


# speedup vs baseline: 2.9118x; 1.0028x over previous
import jax
import jax.numpy as jnp
from jax import lax
from jax.experimental import pallas as pl
from jax.experimental.pallas import tpu as pltpu
from jax.experimental.pallas import tpu_sc as plsc

D = 256
NC, NS = 2, 16
NW = NC * NS
CHUNK = 80
NBUF = 5


def _emb_body(src_tab, trg_tab, src_idx, trg_idx, src_out, trg_out,
              idx_v, rows_v, *sems):
    isem = sems[:NBUF]
    gsem = sems[NBUF:2 * NBUF]
    osem = sems[2 * NBUF:]
    B = src_idx.shape[0]
    bpw = B // NW
    nch = bpw // CHUNK
    nr = nch // NBUF
    wid = lax.axis_index("s") * NC + lax.axis_index("c")
    base = wid * bpw

    for idx_hbm, tab_hbm, out_hbm in ((src_idx, src_tab, src_out),
                                      (trg_idx, trg_tab, trg_out)):

        def istart(b, c):
            off = base + c * CHUNK
            pltpu.async_copy(idx_hbm.at[pl.ds(off, CHUNK)], idx_v.at[b],
                             isem[b])

        def iwait(b, c):
            off = base + c * CHUNK
            pltpu.make_async_copy(idx_hbm.at[pl.ds(off, CHUNK)], idx_v.at[b],
                                  isem[b]).wait()

        def gstart(b):
            pltpu.async_copy(tab_hbm.at[idx_v.at[b]], rows_v.at[b], gsem[b])

        def gwait(b):
            pltpu.make_async_copy(tab_hbm.at[idx_v.at[b]], rows_v.at[b],
                                  gsem[b]).wait()

        def sstart(b, c):
            off = base + c * CHUNK
            pltpu.async_copy(rows_v.at[b], out_hbm.at[pl.ds(off, CHUNK)],
                             osem[b])

        def owait(b, c):
            off = base + c * CHUNK
            pltpu.make_async_copy(rows_v.at[b], out_hbm.at[pl.ds(off, CHUNK)],
                                  osem[b]).wait()

        for b in range(NBUF):
            istart(b, b)
        for b in range(NBUF):
            iwait(b, b)
            gstart(b)
        for b in range(NBUF):
            gwait(b)
            sstart(b, b)
        for b in range(NBUF):
            istart(b, b + NBUF)

        def round_body(r, _):
            g = r * NBUF
            for b in range(NBUF):
                owait(b, g - NBUF + b)
                iwait(b, g + b)
                gstart(b)
            for b in range(NBUF):
                gwait(b)
                sstart(b, g + b)
            for b in range(NBUF):
                istart(b, g + b + NBUF)
            return 0

        lax.fori_loop(1, nr - 1, round_body, 0)

        g = (nr - 1) * NBUF
        for b in range(NBUF):
            owait(b, g - NBUF + b)
            iwait(b, g + b)
            gstart(b)
        for b in range(NBUF):
            gwait(b)
            sstart(b, g + b)
        for b in range(NBUF):
            owait(b, g + b)


def kernel(src_table, trg_table, src_indices, trg_indices):
    Bt, S = src_indices.shape
    B = Bt * S
    si = src_indices.reshape(B)
    ti = trg_indices.reshape(B)
    mesh = plsc.VectorSubcoreMesh(core_axis_name="c", subcore_axis_name="s",
                                  num_cores=NC, num_subcores=NS)
    k = pl.kernel(
        _emb_body,
        out_type=(jax.ShapeDtypeStruct((B, D), jnp.float32),
                  jax.ShapeDtypeStruct((B, D), jnp.float32)),
        mesh=mesh,
        scratch_types=(
            [pltpu.VMEM((NBUF, CHUNK), jnp.int32),
             pltpu.VMEM((NBUF, CHUNK, D), jnp.float32)]
            + [pltpu.SemaphoreType.DMA] * (3 * NBUF)
        ),
    )
    src_out, trg_out = k(src_table, trg_table, si, ti)
    return (src_out.reshape(Bt, S, D), trg_out.reshape(Bt, S, D))


# speedup vs baseline: 2.9121x; 1.0001x over previous
import jax
import jax.numpy as jnp
from jax import lax
from jax.experimental import pallas as pl
from jax.experimental.pallas import tpu as pltpu
from jax.experimental.pallas import tpu_sc as plsc

D = 256
NC, NS = 2, 16
NW = NC * NS
CHUNK = 128
NBUF = 3


def _emb_body(src_tab, trg_tab, src_idx, trg_idx, src_out, trg_out,
              idx_v, rows_v, *sems):
    isem = sems[:NBUF]
    gsem = sems[NBUF:2 * NBUF]
    osem = sems[2 * NBUF:]
    B = src_idx.shape[0]
    bpw = B // NW
    nch = bpw // CHUNK
    nr = nch // NBUF
    wid = lax.axis_index("s") * NC + lax.axis_index("c")
    base = wid * bpw

    for idx_hbm, tab_hbm, out_hbm in ((src_idx, src_tab, src_out),
                                      (trg_idx, trg_tab, trg_out)):

        def istart(b, c):
            off = base + c * CHUNK
            pltpu.async_copy(idx_hbm.at[pl.ds(off, CHUNK)], idx_v.at[b],
                             isem[b])

        def iwait(b, c):
            off = base + c * CHUNK
            pltpu.make_async_copy(idx_hbm.at[pl.ds(off, CHUNK)], idx_v.at[b],
                                  isem[b]).wait()

        def gstart(b):
            pltpu.async_copy(tab_hbm.at[idx_v.at[b]], rows_v.at[b], gsem[b])

        def gwait(b):
            pltpu.make_async_copy(tab_hbm.at[idx_v.at[b]], rows_v.at[b],
                                  gsem[b]).wait()

        def sstart(b, c):
            off = base + c * CHUNK
            pltpu.async_copy(rows_v.at[b], out_hbm.at[pl.ds(off, CHUNK)],
                             osem[b])

        def owait(b, c):
            off = base + c * CHUNK
            pltpu.make_async_copy(rows_v.at[b], out_hbm.at[pl.ds(off, CHUNK)],
                                  osem[b]).wait()

        for b in range(NBUF):
            istart(b, b)
        for b in range(NBUF):
            iwait(b, b)
            gstart(b)
        for b in range(NBUF):
            gwait(b)
            sstart(b, b)
        for b in range(NBUF):
            istart(b, b + NBUF)

        def round_body(r, _):
            g = r * NBUF
            for b in range(NBUF):
                owait(b, g - NBUF + b)
                iwait(b, g + b)
                gstart(b)
            for b in range(NBUF):
                gwait(b)
                sstart(b, g + b)
            for b in range(NBUF):
                istart(b, g + b + NBUF)
            return 0

        lax.fori_loop(1, nr - 1, round_body, 0)

        g = (nr - 1) * NBUF
        for b in range(NBUF):
            owait(b, g - NBUF + b)
            iwait(b, g + b)
            gstart(b)
        for b in range(NBUF):
            gwait(b)
            sstart(b, g + b)
        for b in range(NBUF):
            owait(b, g + b)

        for t in range(nch - nr * NBUF):
            c = nr * NBUF + t
            istart(t, c)
            iwait(t, c)
            gstart(t)
            gwait(t)
            sstart(t, c)
            owait(t, c)


def kernel(src_table, trg_table, src_indices, trg_indices):
    Bt, S = src_indices.shape
    B = Bt * S
    si = src_indices.reshape(B)
    ti = trg_indices.reshape(B)
    mesh = plsc.VectorSubcoreMesh(core_axis_name="c", subcore_axis_name="s",
                                  num_cores=NC, num_subcores=NS)
    k = pl.kernel(
        _emb_body,
        out_type=(jax.ShapeDtypeStruct((B, D), jnp.float32),
                  jax.ShapeDtypeStruct((B, D), jnp.float32)),
        mesh=mesh,
        scratch_types=(
            [pltpu.VMEM((NBUF, CHUNK), jnp.int32),
             pltpu.VMEM((NBUF, CHUNK, D), jnp.float32)]
            + [pltpu.SemaphoreType.DMA] * (3 * NBUF)
        ),
    )
    src_out, trg_out = k(src_table, trg_table, si, ti)
    return (src_out.reshape(Bt, S, D), trg_out.reshape(Bt, S, D))
